# Initial kernel scaffold; baseline (speedup 1.0000x reference)
import jax
import jax.numpy as jnp
from jax.experimental import pallas as pl


def kernel(z, embedding):
    raise NotImplementedError("write your pallas kernel here")


# speedup vs baseline: 1.2152x; 1.2152x over previous
import jax
import jax.numpy as jnp
from jax.experimental import pallas as pl
from jax.experimental.pallas import tpu as pltpu


def _vq_body(z_ref, emb_ref, zq_ref, idx_ref, perp_ref, mdist_ref,
             counts_ref, dsum_ref):
    b = pl.program_id(0)
    nb = pl.num_programs(0)
    zb = z_ref[0]
    emb = emb_ref[...]
    K, C = emb.shape
    HW = zb.shape[1]

    s = jax.lax.dot_general(emb, zb, (((1,), (0,)), ((), ())),
                            preferred_element_type=jnp.float32)
    enorm = jnp.sum(emb * emb, axis=1, keepdims=True)
    znorm = jnp.sum(zb * zb, axis=0, keepdims=True)
    dist = (znorm - 2.0 * s) + enorm

    idx = jnp.argmin(dist, axis=0)
    idx_ref[0, 0, :] = idx

    onehot = (jax.lax.broadcasted_iota(jnp.int32, (K, HW), 0)
              == idx[None, :]).astype(jnp.float32)
    zq = jax.lax.dot_general(emb, onehot, (((0,), (0,)), ((), ())),
                             preferred_element_type=jnp.float32)
    zq_ref[0] = zq

    ones = jnp.ones((1, HW), jnp.float32)
    cnt = jax.lax.dot_general(ones, onehot, (((1,), (1,)), ((), ())),
                              preferred_element_type=jnp.float32)

    @pl.when(b == 0)
    def _init():
        counts_ref[...] = cnt
        dsum_ref[0, 0] = jnp.sum(dist)

    @pl.when(b != 0)
    def _acc():
        counts_ref[...] = counts_ref[...] + cnt
        dsum_ref[0, 0] = dsum_ref[0, 0] + jnp.sum(dist)

    @pl.when(b == nb - 1)
    def _finalize():
        n_total = jnp.float32(nb * HW)
        e_mean = counts_ref[...] / n_total
        perp = jnp.exp(-jnp.sum(e_mean * jnp.log(e_mean + 1e-10)))
        perp_ref[0, 0] = perp
        mdist_ref[0, 0] = dsum_ref[0, 0] / (n_total * jnp.float32(K))


def kernel(z, embedding):
    b, c, h, w = z.shape
    K = embedding.shape[0]
    hw = h * w
    z3 = z.reshape(b, c, hw)

    grid = (b,)
    zq3, idx3, perp, mdist = pl.pallas_call(
        _vq_body,
        grid=grid,
        in_specs=[
            pl.BlockSpec((1, c, hw), lambda i: (i, 0, 0)),
            pl.BlockSpec((K, c), lambda i: (0, 0)),
        ],
        out_specs=[
            pl.BlockSpec((1, c, hw), lambda i: (i, 0, 0)),
            pl.BlockSpec((1, 1, hw), lambda i: (i, 0, 0)),
            pl.BlockSpec(memory_space=pltpu.SMEM),
            pl.BlockSpec(memory_space=pltpu.SMEM),
        ],
        out_shape=[
            jax.ShapeDtypeStruct((b, c, hw), jnp.float32),
            jax.ShapeDtypeStruct((b, 1, hw), jnp.int32),
            jax.ShapeDtypeStruct((1, 1), jnp.float32),
            jax.ShapeDtypeStruct((1, 1), jnp.float32),
        ],
        scratch_shapes=[
            pltpu.VMEM((1, K), jnp.float32),
            pltpu.SMEM((1, 1), jnp.float32),
        ],
    )(z3, embedding)

    z_q = zq3.reshape(b, c, h, w)
    indices = idx3.reshape(b, h, w)
    loss = jnp.zeros((), z.dtype)
    return (z_q, loss, perp.reshape(()), indices, mdist.reshape(()))
